# baseline (speedup 1.0000x reference)
import functools

import jax
import jax.numpy as jnp
from jax import lax
from jax.experimental import pallas as pl
from jax.experimental.pallas import tpu as pltpu
from jax.experimental.pallas import tpu_sc as plsc

B = 16384
VISION_DIM = 768
PROMPT_DIM = 768
NUM_PROMPTS = 1024
TOP_K = 8

BM = 2048

_info = plsc.get_sparse_core_info()
_NC, _NS, _L = _info.num_cores, _info.num_subcores, _info.num_lanes
_NW = _NC * _NS
_RPW = B // _NW
_G = 2
_NCHUNK = _RPW // _G


def _score_body(vf_ref, wt_ref, b_ref, idx_ref, w_ref):
    s0 = (
        jnp.dot(vf_ref[...], wt_ref[...], preferred_element_type=jnp.float32)
        + b_ref[...]
    )
    cols = lax.broadcasted_iota(jnp.int32, (BM, NUM_PROMPTS), 1)
    s = s0
    vals = []
    idxs = []
    for _ in range(TOP_K):
        m = jnp.max(s, axis=1, keepdims=True)
        a = jnp.min(
            jnp.where(s == m, cols, NUM_PROMPTS), axis=1, keepdims=True
        )
        vals.append(m)
        idxs.append(a)
        s = jnp.where(cols == a, -jnp.inf, s)
    v = jnp.concatenate(vals, axis=1)
    a = jnp.concatenate(idxs, axis=1)
    e = jnp.exp(v - v[:, 0:1])
    w = e * (1.0 / jnp.sum(e, axis=1, keepdims=True))
    idx_ref[...] = a
    w_ref[...] = w


def _combine_body(pool_hbm, idx_hbm, w_hbm, out_hbm, idx_v, w_v, rows_v, outb_v, sem):
    wid = lax.axis_index("s") * _NC + lax.axis_index("c")
    base = wid * _RPW
    pltpu.sync_copy(idx_hbm.at[pl.ds(base * TOP_K, _RPW * TOP_K)], idx_v)
    pltpu.sync_copy(
        w_hbm.at[pl.ds(base * TOP_K * _L, _RPW * TOP_K * _L)], w_v
    )

    def chunk(ci, carry):
        pltpu.async_copy(
            pool_hbm.at[idx_v.at[pl.ds(ci * (_G * TOP_K), _G * TOP_K)]],
            rows_v,
            sem,
        ).wait()
        wbase = ci * (_G * TOP_K)
        for g in range(_G):
            ws = []
            for k in range(TOP_K):
                ws.append(w_v[pl.ds((wbase + g * TOP_K + k) * _L, _L)])
            for c in range(PROMPT_DIM // _L):
                acc = ws[0] * rows_v[g * TOP_K + 0, pl.ds(c * _L, _L)]
                for k in range(1, TOP_K):
                    acc = acc + ws[k] * rows_v[g * TOP_K + k, pl.ds(c * _L, _L)]
                outb_v[g, pl.ds(c * _L, _L)] = acc
        pltpu.sync_copy(outb_v, out_hbm.at[pl.ds(base + ci * _G, _G)])
        return carry

    lax.fori_loop(0, _NCHUNK, chunk, 0)


_combine = pl.kernel(
    _combine_body,
    mesh=plsc.VectorSubcoreMesh(core_axis_name="c", subcore_axis_name="s"),
    out_type=jax.ShapeDtypeStruct((B, PROMPT_DIM), jnp.float32),
    scratch_types=[
        pltpu.VMEM((_RPW * TOP_K,), jnp.int32),
        pltpu.VMEM((_RPW * TOP_K * _L,), jnp.float32),
        pltpu.VMEM((_G * TOP_K, PROMPT_DIM), jnp.float32),
        pltpu.VMEM((_G, PROMPT_DIM), jnp.float32),
        pltpu.SemaphoreType.DMA,
    ],
)


@jax.jit
def kernel(vision_features, W, b, prompt_pool):
    wt = W.T
    b2 = b.reshape(1, NUM_PROMPTS)
    grid = (B // BM,)
    idx, w = pl.pallas_call(
        _score_body,
        grid=grid,
        in_specs=[
            pl.BlockSpec((BM, VISION_DIM), lambda i: (i, 0)),
            pl.BlockSpec((VISION_DIM, NUM_PROMPTS), lambda i: (0, 0)),
            pl.BlockSpec((1, NUM_PROMPTS), lambda i: (0, 0)),
        ],
        out_specs=[
            pl.BlockSpec((BM, TOP_K), lambda i: (i, 0)),
            pl.BlockSpec((BM, TOP_K), lambda i: (i, 0)),
        ],
        out_shape=[
            jax.ShapeDtypeStruct((B, TOP_K), jnp.int32),
            jax.ShapeDtypeStruct((B, TOP_K), jnp.float32),
        ],
        compiler_params=pltpu.CompilerParams(
            dimension_semantics=("parallel",),
        ),
    )(vision_features, wt, b2)
    w_exp = jnp.broadcast_to(
        w.reshape(B * TOP_K, 1), (B * TOP_K, _L)
    ).reshape(B * TOP_K * _L)
    return _combine(prompt_pool, idx.reshape(B * TOP_K), w_exp)
